# baseline (speedup 1.0000x reference)
import functools

import jax
import jax.numpy as jnp
from jax import lax
from jax.experimental import pallas as pl
from jax.experimental.pallas import tpu as pltpu
from jax.experimental.pallas import tpu_sc as plsc

N = 10000
D = 128
H = 128
NC = 2
NS = 16
NW = NC * NS
EB = 128
NB = 80
EW_PAD = NB * EB
ROWS_T = 640
NP = NS * ROWS_T
DEG_T = 640
NDEG = NS * DEG_T

_mesh = plsc.VectorSubcoreMesh(
    core_axis_name="c", subcore_axis_name="s", num_cores=NC, num_subcores=NS)


def _deg_body(dst_hbm, ones_hbm, zeros_hbm, out_hbm, dst_v, ones_v, zb, acc1, sem):
    c = lax.axis_index("c")
    s = lax.axis_index("s")
    wid = s * NC + c
    pltpu.sync_copy(zeros_hbm, zb)
    pltpu.sync_copy(zb, acc1.at[pl.ds(s * DEG_T, DEG_T)])
    pltpu.sync_copy(ones_hbm, ones_v)
    pltpu.sync_copy(dst_hbm.at[wid], dst_v)
    plsc.subcore_barrier()

    @pl.loop(0, NB)
    def _(j):
        pltpu.sync_copy(ones_v, acc1.at[dst_v.at[j]], add=True)

    plsc.subcore_barrier()
    pltpu.sync_copy(acc1.at[pl.ds(s * DEG_T, DEG_T)], zb)
    pltpu.sync_copy(zb, out_hbm.at[c, pl.ds(s * DEG_T, DEG_T)])


_deg_kernel = functools.partial(
    pl.kernel,
    out_type=jax.ShapeDtypeStruct((NC, NDEG), jnp.float32),
    mesh=_mesh,
    scratch_types=[
        pltpu.VMEM((NB, EB), jnp.int32),
        pltpu.VMEM((EB,), jnp.float32),
        pltpu.VMEM((DEG_T,), jnp.float32),
        pltpu.MemorySpace.VMEM_SHARED((NDEG,), jnp.float32),
        pltpu.SemaphoreType.DMA,
    ],
)(_deg_body)


_NSLOT = 2
_CH = 16


def _agg_body(g_hbm, src_hbm, dst_hbm, z2_hbm, out_hbm, src_v, dst_v,
              b0, b1, acc, *sems):
    bufs = (b0, b1)
    gsem = sems[:2 * _NSLOT]
    ssem = sems[2 * _NSLOT:]
    c = lax.axis_index("c")
    s = lax.axis_index("s")
    wid = s * NC + c
    pltpu.sync_copy(z2_hbm, b0)
    for k in range(5):
        pltpu.sync_copy(b0, acc.at[pl.ds(s * ROWS_T + k * EB, EB)])
    plsc.subcore_barrier()

    @pl.loop(0, NB // _CH)
    def _(ch):
        pltpu.sync_copy(src_hbm.at[wid, pl.ds(ch * _CH, _CH)], src_v)
        pltpu.sync_copy(dst_hbm.at[wid, pl.ds(ch * _CH, _CH)], dst_v)

        @pl.loop(0, _CH // _NSLOT)
        def _(grp):
            for b in range(_NSLOT):
                j = grp * _NSLOT + b
                for h in range(2):
                    pltpu.async_copy(
                        g_hbm.at[src_v.at[j, pl.ds(h * 64, 64)]],
                        bufs[b].at[pl.ds(h * 64, 64)], gsem[2 * b + h])
            descs = []
            for b in range(_NSLOT):
                j = grp * _NSLOT + b
                for h in range(2):
                    pltpu.make_async_copy(
                        g_hbm.at[src_v.at[j, pl.ds(h * 64, 64)]],
                        bufs[b].at[pl.ds(h * 64, 64)], gsem[2 * b + h]).wait()
                descs.append(pltpu.async_copy(
                    bufs[b], acc.at[dst_v.at[j]], ssem[b], add=True))
            for dsc in descs:
                dsc.wait()

    plsc.subcore_barrier()
    for k in range(5):
        pltpu.sync_copy(acc.at[pl.ds(s * ROWS_T + k * EB, EB)], b0)
        pltpu.sync_copy(b0, out_hbm.at[c, pl.ds(s * ROWS_T + k * EB, EB)])


_agg_kernel = functools.partial(
    pl.kernel,
    out_type=jax.ShapeDtypeStruct((NC, NP, H), jnp.float32),
    mesh=_mesh,
    scratch_types=(
        [pltpu.VMEM((_CH, EB), jnp.int32),
         pltpu.VMEM((_CH, EB), jnp.int32)]
        + [pltpu.VMEM((EB, H), jnp.float32)] * _NSLOT
        + [pltpu.MemorySpace.VMEM_SHARED((NP, H), jnp.float32)]
        + [pltpu.SemaphoreType.DMA] * (3 * _NSLOT)
    ),
)(_agg_body)


_RB = 1000


def _tc_a_body(d0_ref, d1_ref, x_ref, w_ref, g_ref):
    dinv = lax.rsqrt(d0_ref[...] + d1_ref[...] + 1.0)
    g_ref[...] = dinv * jnp.dot(x_ref[...], w_ref[...],
                                preferred_element_type=jnp.float32)


def _tc_b_body(d0_ref, d1_ref, a0_ref, a1_ref, g_ref, b_ref, w_ref, o_ref):
    dinv = lax.rsqrt(d0_ref[...] + d1_ref[...] + 1.0)
    z = dinv * (a0_ref[...] + a1_ref[...] + g_ref[...]) + b_ref[...]
    z = jnp.maximum(z, 0.0)
    o_ref[...] = dinv * jnp.dot(z, w_ref[...], preferred_element_type=jnp.float32)


def _tc_c_body(d0_ref, d1_ref, a0_ref, a1_ref, g_ref, b_ref, o_ref):
    dinv = lax.rsqrt(d0_ref[...] + d1_ref[...] + 1.0)
    o_ref[...] = dinv * (a0_ref[...] + a1_ref[...] + g_ref[...]) + b_ref[...]


_col_spec = pl.BlockSpec((_RB, 1), lambda i: (i, 0))
_row_spec = pl.BlockSpec((_RB, H), lambda i: (i, 0))
_w_spec = pl.BlockSpec((D, H), lambda i: (0, 0))
_b_spec = pl.BlockSpec((1, H), lambda i: (0, 0))
_out_sds = jax.ShapeDtypeStruct((N, H), jnp.float32)
_grid = (N // _RB,)

_tc_a = pl.pallas_call(
    _tc_a_body, grid=_grid,
    in_specs=[_col_spec, _col_spec, _row_spec, _w_spec],
    out_specs=_row_spec, out_shape=_out_sds)

_tc_b = pl.pallas_call(
    _tc_b_body, grid=_grid,
    in_specs=[_col_spec, _col_spec, _row_spec, _row_spec, _row_spec,
              _b_spec, _w_spec],
    out_specs=_row_spec, out_shape=_out_sds)

_tc_c = pl.pallas_call(
    _tc_c_body, grid=_grid,
    in_specs=[_col_spec, _col_spec, _row_spec, _row_spec, _row_spec, _b_spec],
    out_specs=_row_spec, out_shape=_out_sds)


def kernel(x, edge_index, W1, b1, W2, b2):
    E = edge_index.shape[1]
    ew = E // NW
    pad = EW_PAD - ew
    src = edge_index[0].reshape(NW, ew)
    dst = edge_index[1].reshape(NW, ew)
    srcp = jnp.concatenate(
        [src, jnp.zeros((NW, pad), jnp.int32)], axis=1).reshape(NW, NB, EB)
    dstp = jnp.concatenate(
        [dst, jnp.full((NW, pad), N, jnp.int32)], axis=1).reshape(NW, NB, EB)

    ones1 = jnp.ones((EB,), jnp.float32)
    zeros1 = jnp.zeros((DEG_T,), jnp.float32)
    zeros2 = jnp.zeros((EB, H), jnp.float32)

    deg_p = _deg_kernel(dstp, ones1, zeros1)
    d0 = deg_p[0].reshape(NDEG, 1)
    d1 = deg_p[1].reshape(NDEG, 1)

    g1 = _tc_a(d0, d1, x, W1)
    agg1 = _agg_kernel(g1, srcp, dstp, zeros2)
    g2 = _tc_b(d0, d1, agg1[0, :N], agg1[1, :N], g1, b1.reshape(1, H), W2)
    agg2 = _agg_kernel(g2, srcp, dstp, zeros2)
    return _tc_c(d0, d1, agg2[0, :N], agg2[1, :N], g2, b2.reshape(1, H))


# speedup vs baseline: 1.0994x; 1.0994x over previous
import functools

import jax
import jax.numpy as jnp
from jax import lax
from jax.experimental import pallas as pl
from jax.experimental.pallas import tpu as pltpu
from jax.experimental.pallas import tpu_sc as plsc

N = 10000
D = 128
H = 128
NC = 2
NS = 16
NW = NC * NS
EB = 128
NB = 80
EW_PAD = NB * EB
ROWS_T = 640
NP = NS * ROWS_T
DEG_T = 640
NDEG = NS * DEG_T

_mesh = plsc.VectorSubcoreMesh(
    core_axis_name="c", subcore_axis_name="s", num_cores=NC, num_subcores=NS)


def _deg_body(dst_hbm, ones_hbm, zeros_hbm, out_hbm, dst_v, ones_v, zb, acc1, sem):
    c = lax.axis_index("c")
    s = lax.axis_index("s")
    wid = s * NC + c
    pltpu.sync_copy(zeros_hbm, zb)
    pltpu.sync_copy(zb, acc1.at[pl.ds(s * DEG_T, DEG_T)])
    pltpu.sync_copy(ones_hbm, ones_v)
    pltpu.sync_copy(dst_hbm.at[wid], dst_v)
    plsc.subcore_barrier()

    @pl.loop(0, NB)
    def _(j):
        pltpu.sync_copy(ones_v, acc1.at[dst_v.at[j]], add=True)

    plsc.subcore_barrier()
    pltpu.sync_copy(acc1.at[pl.ds(s * DEG_T, DEG_T)], zb)
    pltpu.sync_copy(zb, out_hbm.at[c, pl.ds(s * DEG_T, DEG_T)])


_deg_kernel = functools.partial(
    pl.kernel,
    out_type=jax.ShapeDtypeStruct((NC, NDEG), jnp.float32),
    mesh=_mesh,
    scratch_types=[
        pltpu.VMEM((NB, EB), jnp.int32),
        pltpu.VMEM((EB,), jnp.float32),
        pltpu.VMEM((DEG_T,), jnp.float32),
        pltpu.MemorySpace.VMEM_SHARED((NDEG,), jnp.float32),
        pltpu.SemaphoreType.DMA,
    ],
)(_deg_body)


_NSLOT = 4
_HB = 64
_CH = 32


def _agg_body(g_hbm, src_hbm, dst_hbm, z2_hbm, out_hbm, src_v, dst_v,
              b0, b1, b2, b3, acc, *sems):
    bufs = (b0, b1, b2, b3)
    gsem = sems[:_NSLOT]
    ssem = sems[_NSLOT:]
    c = lax.axis_index("c")
    s = lax.axis_index("s")
    wid = s * NC + c
    pltpu.sync_copy(z2_hbm, b0)
    for k in range(10):
        pltpu.sync_copy(b0, acc.at[pl.ds(s * ROWS_T + k * _HB, _HB)])
    plsc.subcore_barrier()

    @pl.loop(0, 2 * NB // _CH)
    def _(ch):
        pltpu.sync_copy(src_hbm.at[wid, pl.ds(ch * _CH, _CH)], src_v)
        pltpu.sync_copy(dst_hbm.at[wid, pl.ds(ch * _CH, _CH)], dst_v)

        for b in range(_NSLOT):
            pltpu.async_copy(g_hbm.at[src_v.at[b]], bufs[b], gsem[b])

        @pl.loop(0, _CH // _NSLOT - 1)
        def _(grp):
            for b in range(_NSLOT):
                j = grp * _NSLOT + b
                pltpu.make_async_copy(
                    g_hbm.at[src_v.at[j]], bufs[b], gsem[b]).wait()
                pltpu.async_copy(
                    bufs[b], acc.at[dst_v.at[j]], ssem[b], add=True).wait()
                pltpu.async_copy(
                    g_hbm.at[src_v.at[j + _NSLOT]], bufs[b], gsem[b])

        for b in range(_NSLOT):
            j = _CH - _NSLOT + b
            pltpu.make_async_copy(
                g_hbm.at[src_v.at[j]], bufs[b], gsem[b]).wait()
            pltpu.sync_copy(bufs[b], acc.at[dst_v.at[j]], add=True)

    plsc.subcore_barrier()
    for k in range(10):
        pltpu.sync_copy(acc.at[pl.ds(s * ROWS_T + k * _HB, _HB)], b0)
        pltpu.sync_copy(b0, out_hbm.at[c, pl.ds(s * ROWS_T + k * _HB, _HB)])


_agg_kernel = functools.partial(
    pl.kernel,
    out_type=jax.ShapeDtypeStruct((NC, NP, H), jnp.float32),
    mesh=_mesh,
    scratch_types=(
        [pltpu.VMEM((_CH, _HB), jnp.int32),
         pltpu.VMEM((_CH, _HB), jnp.int32)]
        + [pltpu.VMEM((_HB, H), jnp.float32)] * _NSLOT
        + [pltpu.MemorySpace.VMEM_SHARED((NP, H), jnp.float32)]
        + [pltpu.SemaphoreType.DMA] * (2 * _NSLOT)
    ),
)(_agg_body)


_RB = 1000


def _tc_a_body(d0_ref, d1_ref, x_ref, w_ref, g_ref):
    dinv = lax.rsqrt(d0_ref[...] + d1_ref[...] + 1.0)
    g_ref[...] = dinv * jnp.dot(x_ref[...], w_ref[...],
                                preferred_element_type=jnp.float32)


def _tc_b_body(d0_ref, d1_ref, a0_ref, a1_ref, g_ref, b_ref, w_ref, o_ref):
    dinv = lax.rsqrt(d0_ref[...] + d1_ref[...] + 1.0)
    z = dinv * (a0_ref[...] + a1_ref[...] + g_ref[...]) + b_ref[...]
    z = jnp.maximum(z, 0.0)
    o_ref[...] = dinv * jnp.dot(z, w_ref[...], preferred_element_type=jnp.float32)


def _tc_c_body(d0_ref, d1_ref, a0_ref, a1_ref, g_ref, b_ref, o_ref):
    dinv = lax.rsqrt(d0_ref[...] + d1_ref[...] + 1.0)
    o_ref[...] = dinv * (a0_ref[...] + a1_ref[...] + g_ref[...]) + b_ref[...]


_col_spec = pl.BlockSpec((_RB, 1), lambda i: (i, 0))
_row_spec = pl.BlockSpec((_RB, H), lambda i: (i, 0))
_w_spec = pl.BlockSpec((D, H), lambda i: (0, 0))
_b_spec = pl.BlockSpec((1, H), lambda i: (0, 0))
_out_sds = jax.ShapeDtypeStruct((N, H), jnp.float32)
_grid = (N // _RB,)

_tc_a = pl.pallas_call(
    _tc_a_body, grid=_grid,
    in_specs=[_col_spec, _col_spec, _row_spec, _w_spec],
    out_specs=_row_spec, out_shape=_out_sds)

_tc_b = pl.pallas_call(
    _tc_b_body, grid=_grid,
    in_specs=[_col_spec, _col_spec, _row_spec, _row_spec, _row_spec,
              _b_spec, _w_spec],
    out_specs=_row_spec, out_shape=_out_sds)

_tc_c = pl.pallas_call(
    _tc_c_body, grid=_grid,
    in_specs=[_col_spec, _col_spec, _row_spec, _row_spec, _row_spec, _b_spec],
    out_specs=_row_spec, out_shape=_out_sds)


def kernel(x, edge_index, W1, b1, W2, b2):
    E = edge_index.shape[1]
    ew = E // NW
    pad = EW_PAD - ew
    src = edge_index[0].reshape(NW, ew)
    dst = edge_index[1].reshape(NW, ew)
    srcp = jnp.concatenate(
        [src, jnp.zeros((NW, pad), jnp.int32)], axis=1).reshape(NW, NB, EB)
    dstp = jnp.concatenate(
        [dst, jnp.full((NW, pad), N, jnp.int32)], axis=1).reshape(NW, NB, EB)

    ones1 = jnp.ones((EB,), jnp.float32)
    zeros1 = jnp.zeros((DEG_T,), jnp.float32)
    zeros2 = jnp.zeros((_HB, H), jnp.float32)

    deg_p = _deg_kernel(dstp, ones1, zeros1)
    d0 = deg_p[0].reshape(NDEG, 1)
    d1 = deg_p[1].reshape(NDEG, 1)

    g1 = _tc_a(d0, d1, x, W1)
    srch = srcp.reshape(NW, 2 * NB, _HB)
    dsth = dstp.reshape(NW, 2 * NB, _HB)
    agg1 = _agg_kernel(g1, srch, dsth, zeros2)
    g2 = _tc_b(d0, d1, agg1[0, :N], agg1[1, :N], g1, b1.reshape(1, H), W2)
    agg2 = _agg_kernel(g2, srch, dsth, zeros2)
    return _tc_c(d0, d1, agg2[0, :N], agg2[1, :N], g2, b2.reshape(1, H))
